# baseline (device time: 43324 ns/iter reference)
import jax
import jax.numpy as jnp
from jax import lax
from jax.experimental import pallas as pl
from jax.experimental.pallas import tpu as pltpu

N_DEV = 4
N_EXPERTS = 16
CAP = 51


def kernel(x, router_W, route_idx, expert_W):
    n, d = x.shape
    e_loc, _, h = expert_W.shape
    blk = n // N_DEV

    def body(x_ref, ridx_ref, w_ref, out_ref,
             posat_ref, send_buf, recv_buf, send_sems, recv_sems):
        my = lax.axis_index("i")

        route = ridx_ref[:, :]
        onehot = (route == lax.broadcasted_iota(
            jnp.int32, (n, N_EXPERTS), 1)).astype(jnp.float32)
        ri = lax.broadcasted_iota(jnp.int32, (n, n), 0)
        ci = lax.broadcasted_iota(jnp.int32, (n, n), 1)
        tri = (ri >= ci).astype(jnp.float32)
        pos = lax.dot_general(
            tri, onehot, (((1,), (0,)), ((), ())),
            preferred_element_type=jnp.float32)
        posat_ref[:, :] = jnp.sum(pos * onehot, axis=1, keepdims=True)

        def acc_block(dst):
            rows = pl.ds(dst * blk, blk)
            xb = x_ref[rows, :]
            rb = ridx_ref[rows, :]
            pb = posat_ref[rows, :]
            acc = jnp.zeros((blk, h), jnp.float32)
            for le in range(e_loc):
                e = my * e_loc + le
                m = jnp.where((rb == e) & (pb <= CAP), 1.0, 0.0)
                acc = acc + jnp.dot(xb * m, w_ref[le],
                                    preferred_element_type=jnp.float32)
            return acc

        rdmas = []
        for off in (1, 2, 3):
            dst = lax.rem(my + off, N_DEV)
            send_buf[off - 1, :, :] = acc_block(dst)
            rdma = pltpu.make_async_remote_copy(
                src_ref=send_buf.at[off - 1],
                dst_ref=recv_buf.at[off - 1],
                send_sem=send_sems.at[off - 1],
                recv_sem=recv_sems.at[off - 1],
                device_id=(dst,),
                device_id_type=pl.DeviceIdType.MESH,
            )
            rdma.start()
            rdmas.append(rdma)

        own = acc_block(my)

        for rdma in rdmas:
            rdma.wait()
        out_ref[:, :] = (own + recv_buf[0, :, :]
                         + recv_buf[1, :, :] + recv_buf[2, :, :])

    return pl.pallas_call(
        body,
        out_shape=jax.ShapeDtypeStruct((blk, h), jnp.float32),
        in_specs=[
            pl.BlockSpec(memory_space=pltpu.VMEM),
            pl.BlockSpec(memory_space=pltpu.VMEM),
            pl.BlockSpec(memory_space=pltpu.VMEM),
        ],
        out_specs=pl.BlockSpec(memory_space=pltpu.VMEM),
        scratch_shapes=[
            pltpu.VMEM((n, 1), jnp.float32),
            pltpu.VMEM((N_DEV - 1, blk, h), jnp.float32),
            pltpu.VMEM((N_DEV - 1, blk, h), jnp.float32),
            pltpu.SemaphoreType.DMA((N_DEV - 1,)),
            pltpu.SemaphoreType.DMA((N_DEV - 1,)),
        ],
    )(x, route_idx, expert_W)


# device time: 32110 ns/iter; 1.3492x vs baseline; 1.3492x over previous
import jax
import jax.numpy as jnp
from jax import lax
from jax.experimental import pallas as pl
from jax.experimental.pallas import tpu as pltpu

N_DEV = 4
N_EXPERTS = 16
CAP = 51


def kernel(x, router_W, route_idx, expert_W):
    n, d = x.shape
    e_loc, _, h = expert_W.shape
    blk = n // N_DEV

    def body(x_ref, ridx_ref, w_ref, out_ref,
             posat_ref, wbf_ref, send_buf, recv_buf,
             send_sems, recv_sems):
        my = lax.axis_index("i")

        route = ridx_ref[:, :]
        onehot = (route == lax.broadcasted_iota(
            jnp.int32, (n, N_EXPERTS), 1)).astype(jnp.float32)
        ri = lax.broadcasted_iota(jnp.int32, (n, n), 0)
        ci = lax.broadcasted_iota(jnp.int32, (n, n), 1)
        tri = (ri >= ci).astype(jnp.float32)
        pos = lax.dot_general(
            tri, onehot, (((1,), (0,)), ((), ())),
            preferred_element_type=jnp.float32)
        posat_ref[:, :] = jnp.sum(pos * onehot, axis=1, keepdims=True)

        wbf_ref[:, :, :] = w_ref[:, :, :].astype(jnp.bfloat16)

        def acc_block(dst):
            rows = pl.ds(dst * blk, blk)
            xb = x_ref[rows, :]
            rb = ridx_ref[rows, :]
            pb = posat_ref[rows, :]
            acc = jnp.zeros((blk, h), jnp.float32)
            for le in range(e_loc):
                e = my * e_loc + le
                m = jnp.where((rb == e) & (pb <= CAP), 1.0, 0.0)
                xm = (xb * m).astype(jnp.bfloat16)
                acc = acc + jnp.dot(xm, wbf_ref[le],
                                    preferred_element_type=jnp.float32)
            return acc

        rdmas = []
        for off in (1, 2, 3):
            dst = lax.rem(my + off, N_DEV)
            send_buf[off - 1, :, :] = acc_block(dst).astype(jnp.bfloat16)
            rdma = pltpu.make_async_remote_copy(
                src_ref=send_buf.at[off - 1],
                dst_ref=recv_buf.at[off - 1],
                send_sem=send_sems.at[off - 1],
                recv_sem=recv_sems.at[off - 1],
                device_id=(dst,),
                device_id_type=pl.DeviceIdType.MESH,
            )
            rdma.start()
            rdmas.append(rdma)

        own = acc_block(my)

        for rdma in rdmas:
            rdma.wait()
        out_ref[:, :] = (own
                         + recv_buf[0, :, :].astype(jnp.float32)
                         + recv_buf[1, :, :].astype(jnp.float32)
                         + recv_buf[2, :, :].astype(jnp.float32))

    return pl.pallas_call(
        body,
        out_shape=jax.ShapeDtypeStruct((blk, h), jnp.float32),
        in_specs=[
            pl.BlockSpec(memory_space=pltpu.VMEM),
            pl.BlockSpec(memory_space=pltpu.VMEM),
            pl.BlockSpec(memory_space=pltpu.VMEM),
        ],
        out_specs=pl.BlockSpec(memory_space=pltpu.VMEM),
        scratch_shapes=[
            pltpu.VMEM((n, 1), jnp.float32),
            pltpu.VMEM((e_loc, d, h), jnp.bfloat16),
            pltpu.VMEM((N_DEV - 1, blk, h), jnp.bfloat16),
            pltpu.VMEM((N_DEV - 1, blk, h), jnp.bfloat16),
            pltpu.SemaphoreType.DMA((N_DEV - 1,)),
            pltpu.SemaphoreType.DMA((N_DEV - 1,)),
        ],
    )(x, route_idx, expert_W)


# device time: 27696 ns/iter; 1.5643x vs baseline; 1.1594x over previous
import jax
import jax.numpy as jnp
from jax import lax
from jax.experimental import pallas as pl
from jax.experimental.pallas import tpu as pltpu

N_DEV = 4
N_EXPERTS = 16
CAP = 51
CBUF = 128


def kernel(x, router_W, route_idx, expert_W):
    n, d = x.shape
    e_loc, _, h = expert_W.shape
    blk = n // N_DEV

    def body(x_ref, ridx_ref, w_ref, out_ref,
             gk_ref, wbf_ref, send_buf, recv_buf,
             send_sems, recv_sems):
        my = lax.axis_index("i")

        route = ridx_ref[:, :]
        onehot = (route == lax.broadcasted_iota(
            jnp.int32, (n, N_EXPERTS), 1)).astype(jnp.float32)
        ri = lax.broadcasted_iota(jnp.int32, (n, n), 0)
        ci = lax.broadcasted_iota(jnp.int32, (n, n), 1)
        tri = (ri >= ci).astype(jnp.float32)
        pos = lax.dot_general(
            tri, onehot, (((1,), (0,)), ((), ())),
            preferred_element_type=jnp.float32)
        posat = jnp.sum(pos * onehot, axis=1, keepdims=True)
        g = lax.div(route, e_loc).astype(jnp.float32)
        gk_ref[:, :] = jnp.where(posat <= CAP, g, -1.0)

        wbf_ref[:, :, :] = w_ref[:, :, :].astype(jnp.bfloat16)

        rib = lax.broadcasted_iota(jnp.int32, (blk, blk), 0)
        cib = lax.broadcasted_iota(jnp.int32, (blk, blk), 1)
        trib = (rib >= cib).astype(jnp.float32)
        lane = lax.broadcasted_iota(jnp.int32, (blk, CBUF), 1).astype(
            jnp.float32)

        def pt_mat(bstart, q):
            f = jnp.where(gk_ref[pl.ds(bstart, blk), :] == q, 1.0, 0.0)
            c = lax.dot_general(trib, f, (((1,), (0,)), ((), ())),
                                preferred_element_type=jnp.float32) - 1.0
            return ((lane == c) * f).astype(jnp.bfloat16)

        def acc_block(bstart):
            rows = pl.ds(bstart, blk)
            xb = x_ref[rows, :]
            rb = ridx_ref[rows, :]
            acc = jnp.zeros((blk, h), jnp.float32)
            for le in range(e_loc):
                e = my * e_loc + le
                m = jnp.where((rb == e) & (gk_ref[rows, :] >= 0.0), 1.0, 0.0)
                xm = (xb * m).astype(jnp.bfloat16)
                acc = acc + jnp.dot(xm, wbf_ref[le],
                                    preferred_element_type=jnp.float32)
            return acc

        myf = my.astype(jnp.float32)

        rdmas = []
        for off in (1, 2, 3):
            dst = lax.rem(my + off, N_DEV)
            bstart = dst * blk
            acc = acc_block(bstart).astype(jnp.bfloat16)
            pt = pt_mat(bstart, myf)
            send_buf[off - 1, :, :] = lax.dot_general(
                pt, acc, (((0,), (0,)), ((), ())),
                preferred_element_type=jnp.float32).astype(jnp.bfloat16)
            rdma = pltpu.make_async_remote_copy(
                src_ref=send_buf.at[off - 1],
                dst_ref=recv_buf.at[off - 1],
                send_sem=send_sems.at[off - 1],
                recv_sem=recv_sems.at[off - 1],
                device_id=(dst,),
                device_id_type=pl.DeviceIdType.MESH,
            )
            rdma.start()
            rdmas.append(rdma)

        own = acc_block(my * blk)

        pts = []
        for off in (1, 2, 3):
            src = lax.rem(my + N_DEV - off, N_DEV)
            pts.append(pt_mat(my * blk, src.astype(jnp.float32)))

        for off, rdma, pt in zip((1, 2, 3), rdmas, pts):
            rdma.wait()
            own = own + jnp.dot(pt, recv_buf[off - 1, :, :],
                                preferred_element_type=jnp.float32)
        out_ref[:, :] = own

    return pl.pallas_call(
        body,
        out_shape=jax.ShapeDtypeStruct((blk, h), jnp.float32),
        in_specs=[
            pl.BlockSpec(memory_space=pltpu.VMEM),
            pl.BlockSpec(memory_space=pltpu.VMEM),
            pl.BlockSpec(memory_space=pltpu.VMEM),
        ],
        out_specs=pl.BlockSpec(memory_space=pltpu.VMEM),
        scratch_shapes=[
            pltpu.VMEM((n, 1), jnp.float32),
            pltpu.VMEM((e_loc, d, h), jnp.bfloat16),
            pltpu.VMEM((N_DEV - 1, CBUF, h), jnp.bfloat16),
            pltpu.VMEM((N_DEV - 1, CBUF, h), jnp.bfloat16),
            pltpu.SemaphoreType.DMA((N_DEV - 1,)),
            pltpu.SemaphoreType.DMA((N_DEV - 1,)),
        ],
    )(x, route_idx, expert_W)


# device time: 25873 ns/iter; 1.6745x vs baseline; 1.0705x over previous
import jax
import jax.numpy as jnp
from jax import lax
from jax.experimental import pallas as pl
from jax.experimental.pallas import tpu as pltpu

N_DEV = 4
N_EXPERTS = 16
CAP = 51
CSLOT = 64
CBUF = 128


def kernel(x, router_W, route_idx, expert_W):
    n, d = x.shape
    e_loc, _, h = expert_W.shape
    blk = n // N_DEV
    nslot = e_loc * CSLOT

    def body(x_ref, ridx_ref, w_ref, out_ref,
             gk_ref, slot_ref, wbf_ref, y_ref, send_buf, recv_buf,
             send_sems, recv_sems):
        my = lax.axis_index("i")
        myf = my.astype(jnp.float32)

        route = ridx_ref[:, :]
        onehot = (route == lax.broadcasted_iota(
            jnp.int32, (n, N_EXPERTS), 1)).astype(jnp.float32)
        ri = lax.broadcasted_iota(jnp.int32, (n, n), 0)
        ci = lax.broadcasted_iota(jnp.int32, (n, n), 1)
        tri = (ri >= ci).astype(jnp.float32)
        pos = lax.dot_general(
            tri, onehot, (((1,), (0,)), ((), ())),
            preferred_element_type=jnp.float32)
        posat = jnp.sum(pos * onehot, axis=1, keepdims=True)
        g = lax.div(route, e_loc).astype(jnp.float32)
        gk_ref[:, :] = jnp.where(posat <= CAP, g, -1.0)
        routef = route.astype(jnp.float32)
        slot_ref[:, :] = (routef - e_loc * g) * CSLOT + posat - 1.0

        wbf_ref[:, :, :] = w_ref[:, :, :].astype(jnp.bfloat16)

        lane_s = lax.broadcasted_iota(jnp.int32, (n, nslot), 1).astype(
            jnp.float32)
        vmine = jnp.where(gk_ref[:, :] == myf, 1.0, 0.0)
        gt = ((lane_s == slot_ref[:, :]) * vmine).astype(jnp.bfloat16)
        xg = lax.dot_general(
            gt, x_ref[:, :].astype(jnp.bfloat16), (((0,), (0,)), ((), ())),
            preferred_element_type=jnp.float32).astype(jnp.bfloat16)
        for le in range(e_loc):
            y_ref[le * CSLOT:(le + 1) * CSLOT, :] = jnp.dot(
                xg[le * CSLOT:(le + 1) * CSLOT, :], wbf_ref[le],
                preferred_element_type=jnp.float32).astype(jnp.bfloat16)
        yv = y_ref[:, :]

        rib = lax.broadcasted_iota(jnp.int32, (blk, blk), 0)
        cib = lax.broadcasted_iota(jnp.int32, (blk, blk), 1)
        trib = (rib >= cib).astype(jnp.float32)
        lane_c = lax.broadcasted_iota(jnp.int32, (blk, CBUF), 1).astype(
            jnp.float32)
        lane_sb = lax.broadcasted_iota(jnp.int32, (blk, nslot), 1).astype(
            jnp.float32)

        def pt_mat(bstart, q):
            f = jnp.where(gk_ref[pl.ds(bstart, blk), :] == q, 1.0, 0.0)
            c = lax.dot_general(trib, f, (((1,), (0,)), ((), ())),
                                preferred_element_type=jnp.float32) - 1.0
            return ((lane_c == c) * f).astype(jnp.bfloat16)

        rdmas = []
        for off in (1, 2, 3):
            dst = lax.rem(my + off, N_DEV)
            bstart = dst * blk
            rows = pl.ds(bstart, blk)
            fb = jnp.where(gk_ref[rows, :] == myf, 1.0, 0.0)
            sd = ((lane_sb == slot_ref[rows, :]) * fb).astype(
                jnp.bfloat16)
            pt = pt_mat(bstart, myf)
            qd = lax.dot_general(
                pt, sd, (((0,), (0,)), ((), ())),
                preferred_element_type=jnp.float32).astype(jnp.bfloat16)
            send_buf[off - 1, :, :] = jnp.dot(
                qd, yv, preferred_element_type=jnp.float32).astype(
                jnp.bfloat16)
            rdma = pltpu.make_async_remote_copy(
                src_ref=send_buf.at[off - 1],
                dst_ref=recv_buf.at[off - 1],
                send_sem=send_sems.at[off - 1],
                recv_sem=recv_sems.at[off - 1],
                device_id=(dst,),
                device_id_type=pl.DeviceIdType.MESH,
            )
            rdma.start()
            rdmas.append(rdma)

        rows = pl.ds(my * blk, blk)
        fb = jnp.where(gk_ref[rows, :] == myf, 1.0, 0.0)
        s_my = ((lane_sb == slot_ref[rows, :]) * fb).astype(jnp.bfloat16)
        own = jnp.dot(s_my, yv, preferred_element_type=jnp.float32)

        pts = []
        for off in (1, 2, 3):
            src = lax.rem(my + N_DEV - off, N_DEV)
            pts.append(pt_mat(my * blk, src.astype(jnp.float32)))

        for off, rdma, pt in zip((1, 2, 3), rdmas, pts):
            rdma.wait()
            own = own + jnp.dot(pt, recv_buf[off - 1, :, :],
                                preferred_element_type=jnp.float32)
        out_ref[:, :] = own

    return pl.pallas_call(
        body,
        out_shape=jax.ShapeDtypeStruct((blk, h), jnp.float32),
        in_specs=[
            pl.BlockSpec(memory_space=pltpu.VMEM),
            pl.BlockSpec(memory_space=pltpu.VMEM),
            pl.BlockSpec(memory_space=pltpu.VMEM),
        ],
        out_specs=pl.BlockSpec(memory_space=pltpu.VMEM),
        scratch_shapes=[
            pltpu.VMEM((n, 1), jnp.float32),
            pltpu.VMEM((n, 1), jnp.float32),
            pltpu.VMEM((e_loc, d, h), jnp.bfloat16),
            pltpu.VMEM((e_loc * CSLOT, h), jnp.bfloat16),
            pltpu.VMEM((N_DEV - 1, CBUF, h), jnp.bfloat16),
            pltpu.VMEM((N_DEV - 1, CBUF, h), jnp.bfloat16),
            pltpu.SemaphoreType.DMA((N_DEV - 1,)),
            pltpu.SemaphoreType.DMA((N_DEV - 1,)),
        ],
    )(x, route_idx, expert_W)
